# baseline (speedup 1.0000x reference)
import functools

import jax
import jax.numpy as jnp
from jax import lax
from jax.experimental import pallas as pl
from jax.experimental.pallas import tpu as pltpu
from jax.experimental.pallas import tpu_sc as plsc

N = 10000
E = 320000
D = 128
D_PAD = 144
NC = 2
NS = 16
NW = NC * NS
CH = 128
G_TOT = 160
SPLIT0 = 112
PC = 2
E_PAD = NS * G_TOT * CH
N_PAD = 10240
ROWS_PER_TILE = N_PAD // NS


def _sc_kernel(xp_hbm, sd_hbm, z_hbm, parts_hbm,
               sdidx, rows, acc, sems):
    c = lax.axis_index("c")
    s = lax.axis_index("s")
    grps = lax.select(c == 0, jnp.int32(SPLIT0), jnp.int32(G_TOT - SPLIT0))
    pieces = grps // PC

    gbase = s * G_TOT + c * SPLIT0

    def issue_idx_piece(p, buf):
        pltpu.async_copy(sd_hbm.at[pl.ds(gbase + p * PC, PC)],
                         sdidx.at[buf], sems[2 + buf])

    def wait_idx_piece(buf):
        pltpu.make_async_copy(sd_hbm.at[pl.ds(0, PC)], sdidx.at[buf],
                              sems[2 + buf]).wait()

    def issue_gather(pbuf, j, buf):
        pltpu.async_copy(xp_hbm.at[sdidx.at[pbuf, j, 0]], rows.at[buf],
                         sems[buf])

    def wait_gather(pbuf, j, buf):
        pltpu.make_async_copy(xp_hbm.at[sdidx.at[pbuf, j, 0]],
                              rows.at[buf], sems[buf]).wait()

    rbase = s * ROWS_PER_TILE
    issue_idx_piece(0, 0)
    pltpu.sync_copy(z_hbm, acc.at[pl.ds(rbase, ROWS_PER_TILE)])
    wait_idx_piece(0)
    issue_gather(0, 0, 0)
    issue_idx_piece(1, 1)
    plsc.subcore_barrier()

    def body(p2, _):
        for pp in range(2):
            p = p2 * 2 + pp
            pb = pp
            npb = 1 - pp
            for j in range(PC):
                b = j % 2
                nb = (j + 1) % 2

                if j + 1 < PC:
                    issue_gather(pb, j + 1, nb)
                else:
                    @pl.when(p + 1 < pieces)
                    def _():
                        wait_idx_piece(npb)
                        issue_gather(npb, 0, nb)

                wait_gather(pb, j, b)
                pltpu.sync_copy(rows.at[b], acc.at[sdidx.at[pb, j, 1]],
                                add=True)

                if j == PC - 1:
                    @pl.when(p + 2 < pieces)
                    def _():
                        issue_idx_piece(p + 2, pb)
        return ()

    lax.fori_loop(0, pieces // 2, body, (), unroll=False)
    plsc.subcore_barrier()

    pltpu.sync_copy(acc.at[pl.ds(rbase, ROWS_PER_TILE)],
                    parts_hbm.at[c, pl.ds(rbase, ROWS_PER_TILE)])


_sc_call = functools.partial(
    pl.kernel,
    out_type=jax.ShapeDtypeStruct((NC, N_PAD, D_PAD), jnp.float32),
    mesh=plsc.VectorSubcoreMesh(core_axis_name="c", subcore_axis_name="s"),
    scratch_types=[
        pltpu.VMEM((2, PC, 2, CH), jnp.int32),
        pltpu.VMEM((2, CH, D_PAD), jnp.float32),
        pltpu.VMEM_SHARED((N_PAD, D_PAD), jnp.float32),
        [pltpu.SemaphoreType.DMA] * 4,
    ],
    compiler_params=pltpu.CompilerParams(use_tc_tiling_on_sc=False),
)(_sc_kernel)


def _tc_kernel(x_ref, parts_ref, w_ref, b_ref, a_ref, o_ref):
    p = parts_ref[0] + parts_ref[1]
    agg = x_ref[...] + p[:, :D]
    deg = p[:, D:D + 1] + 1.0
    h = jnp.dot(agg / deg, w_ref[...],
                preferred_element_type=jnp.float32) + b_ref[...]
    o_ref[...] = jnp.where(h >= 0.0, h, a_ref[...] * h)


def _tc_call(x, parts, W, b2, a2):
    B = 400
    grid = (N // B,)
    return pl.pallas_call(
        _tc_kernel,
        grid=grid,
        in_specs=[
            pl.BlockSpec((B, D), lambda i: (i, 0)),
            pl.BlockSpec((NC, B, D_PAD), lambda i: (0, i, 0)),
            pl.BlockSpec((D, D), lambda i: (0, 0)),
            pl.BlockSpec((1, D), lambda i: (0, 0)),
            pl.BlockSpec((1, D), lambda i: (0, 0)),
        ],
        out_specs=pl.BlockSpec((B, D), lambda i: (i, 0)),
        out_shape=jax.ShapeDtypeStruct((N, D), jnp.float32),
    )(x, parts, W, b2, a2)


@jax.jit
def kernel(x, edge_index, W, b, prelu_a):
    src = edge_index[0]
    dst = edge_index[1]
    xp = jnp.zeros((N_PAD, D_PAD), jnp.float32)
    xp = xp.at[:N, :D].set(x).at[:N, D].set(1.0)
    srcp = jnp.full((E_PAD,), N_PAD - 1, jnp.int32).at[:E].set(src)
    dstp = jnp.full((E_PAD,), N_PAD - 1, jnp.int32).at[:E].set(dst)
    sd = jnp.stack([srcp.reshape(E_PAD // CH, CH),
                    dstp.reshape(E_PAD // CH, CH)], axis=1)
    z = jnp.zeros((ROWS_PER_TILE, D_PAD), jnp.float32)

    parts = _sc_call(xp, sd, z)

    b2 = b.reshape(1, D)
    a2 = jnp.broadcast_to(prelu_a.reshape(1, 1), (1, D))
    return _tc_call(x, parts, W, b2, a2)


# speedup vs baseline: 1.0892x; 1.0892x over previous
import functools

import jax
import jax.numpy as jnp
from jax import lax
from jax.experimental import pallas as pl
from jax.experimental.pallas import tpu as pltpu
from jax.experimental.pallas import tpu_sc as plsc

N = 10000
E = 320000
D = 128
D_PAD = 144
NC = 2
NS = 16
NW = NC * NS
CH = 128
G_TOT = 160
SPLIT0 = 120
PC = 2
E_PAD = NS * G_TOT * CH
N_PAD = 10240
ROWS_PER_TILE = N_PAD // NS


def _sc_kernel(xp_hbm, src_hbm, dst_hbm, z_hbm, parts_hbm,
               sidx, didx, rows, acc, sems):
    c = lax.axis_index("c")
    s = lax.axis_index("s")
    grps = lax.select(c == 0, jnp.int32(SPLIT0), jnp.int32(G_TOT - SPLIT0))
    pieces = grps // PC

    gbase = s * G_TOT + c * SPLIT0

    def issue_idx_piece(p, buf):
        pltpu.async_copy(src_hbm.at[pl.ds(gbase + p * PC, PC)],
                         sidx.at[buf], sems[2 + buf])
        pltpu.async_copy(dst_hbm.at[pl.ds(gbase + p * PC, PC)],
                         didx.at[buf], sems[2 + buf])

    def wait_idx_piece(buf):
        pltpu.make_async_copy(src_hbm.at[pl.ds(0, PC)], sidx.at[buf],
                              sems[2 + buf]).wait()
        pltpu.make_async_copy(dst_hbm.at[pl.ds(0, PC)], didx.at[buf],
                              sems[2 + buf]).wait()

    def issue_gather(pbuf, j, buf):
        pltpu.async_copy(xp_hbm.at[sidx.at[pbuf, j]], rows.at[buf],
                         sems[buf])

    def wait_gather(pbuf, j, buf):
        pltpu.make_async_copy(xp_hbm.at[sidx.at[pbuf, j]], rows.at[buf],
                              sems[buf]).wait()

    rbase = s * ROWS_PER_TILE
    issue_idx_piece(0, 0)
    pltpu.sync_copy(z_hbm, acc.at[pl.ds(rbase, ROWS_PER_TILE)])
    wait_idx_piece(0)
    issue_gather(0, 0, 0)
    issue_idx_piece(1, 1)
    plsc.subcore_barrier()

    def body(p2, _):
        for pp in range(2):
            p = p2 * 2 + pp
            pb = pp
            npb = 1 - pp
            for j in range(PC):
                b = j % 2
                nb = (j + 1) % 2

                if j + 1 < PC:
                    issue_gather(pb, j + 1, nb)
                else:
                    @pl.when(p + 1 < pieces)
                    def _():
                        wait_idx_piece(npb)
                        issue_gather(npb, 0, nb)

                wait_gather(pb, j, b)
                pltpu.sync_copy(rows.at[b], acc.at[didx.at[pb, j]],
                                add=True)

                if j == PC - 1:
                    @pl.when(p + 2 < pieces)
                    def _():
                        issue_idx_piece(p + 2, pb)
        return ()

    lax.fori_loop(0, pieces // 2, body, (), unroll=False)
    plsc.subcore_barrier()

    pltpu.sync_copy(acc.at[pl.ds(rbase, ROWS_PER_TILE)],
                    parts_hbm.at[c, pl.ds(rbase, ROWS_PER_TILE)])


_sc_call = functools.partial(
    pl.kernel,
    out_type=jax.ShapeDtypeStruct((NC, N_PAD, D_PAD), jnp.float32),
    mesh=plsc.VectorSubcoreMesh(core_axis_name="c", subcore_axis_name="s"),
    scratch_types=[
        pltpu.VMEM((2, PC, CH), jnp.int32),
        pltpu.VMEM((2, PC, CH), jnp.int32),
        pltpu.VMEM((2, CH, D_PAD), jnp.float32),
        pltpu.VMEM_SHARED((N_PAD, D_PAD), jnp.float32),
        [pltpu.SemaphoreType.DMA] * 4,
    ],
    compiler_params=pltpu.CompilerParams(use_tc_tiling_on_sc=False),
)(_sc_kernel)


def _tc_kernel(x_ref, parts_ref, w_ref, b_ref, a_ref, o_ref):
    p = parts_ref[0] + parts_ref[1]
    agg = x_ref[...] + p[:, :D]
    deg = p[:, D:D + 1] + 1.0
    h = jnp.dot(agg / deg, w_ref[...],
                preferred_element_type=jnp.float32) + b_ref[...]
    o_ref[...] = jnp.where(h >= 0.0, h, a_ref[...] * h)


def _tc_call(x, parts, W, b2, a2):
    B = 400
    grid = (N // B,)
    return pl.pallas_call(
        _tc_kernel,
        grid=grid,
        in_specs=[
            pl.BlockSpec((B, D), lambda i: (i, 0)),
            pl.BlockSpec((NC, B, D_PAD), lambda i: (0, i, 0)),
            pl.BlockSpec((D, D), lambda i: (0, 0)),
            pl.BlockSpec((1, D), lambda i: (0, 0)),
            pl.BlockSpec((1, D), lambda i: (0, 0)),
        ],
        out_specs=pl.BlockSpec((B, D), lambda i: (i, 0)),
        out_shape=jax.ShapeDtypeStruct((N, D), jnp.float32),
    )(x, parts, W, b2, a2)


@jax.jit
def kernel(x, edge_index, W, b, prelu_a):
    src = edge_index[0]
    dst = edge_index[1]
    xp = jnp.zeros((N_PAD, D_PAD), jnp.float32)
    xp = xp.at[:N, :D].set(x).at[:N, D].set(1.0)
    srcp = jnp.full((E_PAD,), N_PAD - 1, jnp.int32).at[:E].set(src)
    dstp = jnp.full((E_PAD,), N_PAD - 1, jnp.int32).at[:E].set(dst)
    srcp = srcp.reshape(E_PAD // CH, CH)
    dstp = dstp.reshape(E_PAD // CH, CH)
    z = jnp.zeros((ROWS_PER_TILE, D_PAD), jnp.float32)

    parts = _sc_call(xp, srcp, dstp, z)

    b2 = b.reshape(1, D)
    a2 = jnp.broadcast_to(prelu_a.reshape(1, 1), (1, D))
    return _tc_call(x, parts, W, b2, a2)


# speedup vs baseline: 1.0907x; 1.0014x over previous
import functools

import jax
import jax.numpy as jnp
from jax import lax
from jax.experimental import pallas as pl
from jax.experimental.pallas import tpu as pltpu
from jax.experimental.pallas import tpu_sc as plsc

N = 10000
E = 320000
D = 128
D_PAD = 144
NC = 2
NS = 16
NW = NC * NS
CH = 128
G_TOT = 160
SPLIT0 = 128
PC = 2
E_PAD = NS * G_TOT * CH
N_PAD = 10240
ROWS_PER_TILE = N_PAD // NS


def _sc_kernel(xp_hbm, src_hbm, dst_hbm, z_hbm, parts_hbm,
               sidx, didx, rows, acc, sems):
    c = lax.axis_index("c")
    s = lax.axis_index("s")
    grps = lax.select(c == 0, jnp.int32(SPLIT0), jnp.int32(G_TOT - SPLIT0))
    pieces = grps // PC

    gbase = s * G_TOT + c * SPLIT0

    def issue_idx_piece(p, buf):
        pltpu.async_copy(src_hbm.at[pl.ds(gbase + p * PC, PC)],
                         sidx.at[buf], sems[2 + buf])
        pltpu.async_copy(dst_hbm.at[pl.ds(gbase + p * PC, PC)],
                         didx.at[buf], sems[2 + buf])

    def wait_idx_piece(buf):
        pltpu.make_async_copy(src_hbm.at[pl.ds(0, PC)], sidx.at[buf],
                              sems[2 + buf]).wait()
        pltpu.make_async_copy(dst_hbm.at[pl.ds(0, PC)], didx.at[buf],
                              sems[2 + buf]).wait()

    def issue_gather(pbuf, j, buf):
        pltpu.async_copy(xp_hbm.at[sidx.at[pbuf, j]], rows.at[buf],
                         sems[buf])

    def wait_gather(pbuf, j, buf):
        pltpu.make_async_copy(xp_hbm.at[sidx.at[pbuf, j]], rows.at[buf],
                              sems[buf]).wait()

    rbase = s * ROWS_PER_TILE
    issue_idx_piece(0, 0)
    pltpu.sync_copy(z_hbm, acc.at[pl.ds(rbase, ROWS_PER_TILE)])
    wait_idx_piece(0)
    issue_gather(0, 0, 0)
    issue_idx_piece(1, 1)
    plsc.subcore_barrier()

    def body(p2, _):
        for pp in range(2):
            p = p2 * 2 + pp
            pb = pp
            npb = 1 - pp
            for j in range(PC):
                b = j % 2
                nb = (j + 1) % 2

                if j + 1 < PC:
                    issue_gather(pb, j + 1, nb)
                else:
                    @pl.when(p + 1 < pieces)
                    def _():
                        wait_idx_piece(npb)
                        issue_gather(npb, 0, nb)

                wait_gather(pb, j, b)
                pltpu.sync_copy(rows.at[b], acc.at[didx.at[pb, j]],
                                add=True)

                if j == PC - 1:
                    @pl.when(p + 2 < pieces)
                    def _():
                        issue_idx_piece(p + 2, pb)
        return ()

    lax.fori_loop(0, pieces // 2, body, (), unroll=False)
    plsc.subcore_barrier()

    pltpu.sync_copy(acc.at[pl.ds(rbase, ROWS_PER_TILE)],
                    parts_hbm.at[c, pl.ds(rbase, ROWS_PER_TILE)])


_sc_call = functools.partial(
    pl.kernel,
    out_type=jax.ShapeDtypeStruct((NC, N_PAD, D_PAD), jnp.float32),
    mesh=plsc.VectorSubcoreMesh(core_axis_name="c", subcore_axis_name="s"),
    scratch_types=[
        pltpu.VMEM((2, PC, CH), jnp.int32),
        pltpu.VMEM((2, PC, CH), jnp.int32),
        pltpu.VMEM((2, CH, D_PAD), jnp.float32),
        pltpu.VMEM_SHARED((N_PAD, D_PAD), jnp.float32),
        [pltpu.SemaphoreType.DMA] * 4,
    ],
    compiler_params=pltpu.CompilerParams(use_tc_tiling_on_sc=False),
)(_sc_kernel)


def _tc_kernel(x_ref, parts_ref, w_ref, b_ref, a_ref, o_ref):
    p = parts_ref[0] + parts_ref[1]
    agg = x_ref[...] + p[:, :D]
    deg = p[:, D:D + 1] + 1.0
    h = jnp.dot(agg / deg, w_ref[...],
                preferred_element_type=jnp.float32) + b_ref[...]
    o_ref[...] = jnp.where(h >= 0.0, h, a_ref[...] * h)


def _tc_call(x, parts, W, b2, a2):
    B = 400
    grid = (N // B,)
    return pl.pallas_call(
        _tc_kernel,
        grid=grid,
        in_specs=[
            pl.BlockSpec((B, D), lambda i: (i, 0)),
            pl.BlockSpec((NC, B, D_PAD), lambda i: (0, i, 0)),
            pl.BlockSpec((D, D), lambda i: (0, 0)),
            pl.BlockSpec((1, D), lambda i: (0, 0)),
            pl.BlockSpec((1, D), lambda i: (0, 0)),
        ],
        out_specs=pl.BlockSpec((B, D), lambda i: (i, 0)),
        out_shape=jax.ShapeDtypeStruct((N, D), jnp.float32),
    )(x, parts, W, b2, a2)


@jax.jit
def kernel(x, edge_index, W, b, prelu_a):
    src = edge_index[0]
    dst = edge_index[1]
    xp = jnp.zeros((N_PAD, D_PAD), jnp.float32)
    xp = xp.at[:N, :D].set(x).at[:N, D].set(1.0)
    srcp = jnp.full((E_PAD,), N_PAD - 1, jnp.int32).at[:E].set(src)
    dstp = jnp.full((E_PAD,), N_PAD - 1, jnp.int32).at[:E].set(dst)
    srcp = srcp.reshape(E_PAD // CH, CH)
    dstp = dstp.reshape(E_PAD // CH, CH)
    z = jnp.zeros((ROWS_PER_TILE, D_PAD), jnp.float32)

    parts = _sc_call(xp, srcp, dstp, z)

    b2 = b.reshape(1, D)
    a2 = jnp.broadcast_to(prelu_a.reshape(1, 1), (1, D))
    return _tc_call(x, parts, W, b2, a2)
